# baseline (speedup 1.0000x reference)
import jax
import jax.numpy as jnp
from jax.experimental import pallas as pl
from jax.experimental.pallas import tpu as pltpu

B, N, M, D = 16, 2048, 2048, 3
CHUNK = 256


def _chamfer_body(pred_ref, gtt_ref, out1_ref, out2_ref):
    pred = pred_ref[0]
    gtt = gtt_ref[0]
    g2 = jnp.sum(gtt * gtt, axis=0, keepdims=True)
    ones_row = jnp.ones((1, M), dtype=jnp.float32)
    bmat = jnp.concatenate([-2.0 * gtt, ones_row, g2], axis=0)
    colmin = jnp.full((1, M), jnp.inf, dtype=jnp.float32)
    sum1 = jnp.float32(0.0)
    for c in range(N // CHUNK):
        pc = pred[c * CHUNK:(c + 1) * CHUNK, :]
        p2 = jnp.sum(pc * pc, axis=1, keepdims=True)
        ones_col = jnp.ones((CHUNK, 1), dtype=jnp.float32)
        amat = jnp.concatenate([pc, p2, ones_col], axis=1)
        d = jax.lax.dot_general(
            amat, bmat, (((1,), (0,)), ((), ())),
            precision=jax.lax.Precision.HIGHEST,
            preferred_element_type=jnp.float32)
        rmin = jnp.min(d, axis=1)
        sum1 = sum1 + jnp.sum(jnp.maximum(rmin, 0.0))
        colmin = jnp.minimum(colmin, jnp.min(d, axis=0, keepdims=True))
    sum2 = jnp.sum(jnp.maximum(colmin, 0.0))
    out1_ref[0, 0, 0] = sum1
    out2_ref[0, 0, 0] = sum2


def kernel(pred, gt):
    gtt = jnp.transpose(gt, (0, 2, 1))
    s1, s2 = pl.pallas_call(
        _chamfer_body,
        grid=(B,),
        in_specs=[
            pl.BlockSpec((1, N, D), lambda b: (b, 0, 0)),
            pl.BlockSpec((1, D, M), lambda b: (b, 0, 0)),
        ],
        out_specs=[
            pl.BlockSpec((1, 1, 1), lambda b: (b, 0, 0),
                         memory_space=pltpu.SMEM),
            pl.BlockSpec((1, 1, 1), lambda b: (b, 0, 0),
                         memory_space=pltpu.SMEM),
        ],
        out_shape=[
            jax.ShapeDtypeStruct((B, 1, 1), jnp.float32),
            jax.ShapeDtypeStruct((B, 1, 1), jnp.float32),
        ],
        compiler_params=pltpu.CompilerParams(
            dimension_semantics=("parallel",)),
    )(pred, gtt)
    return s1.sum() / (B * N) + s2.sum() / (B * M)


# speedup vs baseline: 3.7731x; 3.7731x over previous
import jax
import jax.numpy as jnp
from jax.experimental import pallas as pl
from jax.experimental.pallas import tpu as pltpu

B, N, M, D = 16, 2048, 2048, 3
CHUNK = 256


def _chamfer_body(pred_ref, gtt2_ref, out1_ref, out2_ref):
    pred = pred_ref[0]
    gtt2 = gtt2_ref[0]
    g2 = 0.25 * jnp.sum(gtt2 * gtt2, axis=0, keepdims=True)
    bmat = jnp.concatenate([gtt2, g2], axis=0)
    colmin = jnp.full((1, M), jnp.inf, dtype=jnp.float32)
    sum1 = jnp.float32(0.0)
    for c in range(N // CHUNK):
        pc = pred[c * CHUNK:(c + 1) * CHUNK, :]
        p2 = jnp.sum(pc * pc, axis=1, keepdims=True)
        ones_c = jnp.ones((CHUNK, 1), dtype=jnp.float32)
        amat = jnp.concatenate([pc, ones_c], axis=1)
        d = jax.lax.dot_general(
            amat, bmat, (((1,), (0,)), ((), ())),
            preferred_element_type=jnp.float32) + p2
        rmin = jnp.min(d, axis=1)
        sum1 = sum1 + jnp.sum(jnp.maximum(rmin, 0.0))
        colmin = jnp.minimum(colmin, jnp.min(d, axis=0, keepdims=True))
    sum2 = jnp.sum(jnp.maximum(colmin, 0.0))
    out1_ref[0, 0, 0] = sum1
    out2_ref[0, 0, 0] = sum2


def kernel(pred, gt):
    gtt2 = -2.0 * jnp.transpose(gt, (0, 2, 1))
    s1, s2 = pl.pallas_call(
        _chamfer_body,
        grid=(B,),
        in_specs=[
            pl.BlockSpec((1, N, D), lambda b: (b, 0, 0)),
            pl.BlockSpec((1, D, M), lambda b: (b, 0, 0)),
        ],
        out_specs=[
            pl.BlockSpec((1, 1, 1), lambda b: (b, 0, 0),
                         memory_space=pltpu.SMEM),
            pl.BlockSpec((1, 1, 1), lambda b: (b, 0, 0),
                         memory_space=pltpu.SMEM),
        ],
        out_shape=[
            jax.ShapeDtypeStruct((B, 1, 1), jnp.float32),
            jax.ShapeDtypeStruct((B, 1, 1), jnp.float32),
        ],
        compiler_params=pltpu.CompilerParams(
            dimension_semantics=("parallel",)),
    )(pred, gtt2)
    return s1.sum() / (B * N) + s2.sum() / (B * M)
